# baseline (speedup 1.0000x reference)
import jax
import jax.numpy as jnp
from jax.experimental import pallas as pl
from jax.experimental.pallas import tpu as pltpu

TOPK = 64
_NT = 512
_NBINS = 512
_P = 4


def _enc_tile(x_ref, we_ref, be_ref):
    xb = x_ref[...]
    wb = we_ref[...].astype(jnp.bfloat16)
    acc = jnp.dot(xb, wb, preferred_element_type=jnp.float32)
    return jnp.maximum(acc + be_ref[...], 0.0)


def _tau_kernel(x_ref, we_ref, be_ref, tau_ref, *m_refs):
    i = pl.program_id(0)
    nsteps = pl.num_programs(0)
    act = _enc_tile(x_ref, we_ref, be_ref)

    @pl.when(i == 0)
    def _():
        for r in m_refs:
            r[...] = jnp.zeros_like(r)

    for s in range(_NT // _NBINS):
        t = act[:, s * _NBINS:(s + 1) * _NBINS]
        for r in m_refs:
            cur = r[...]
            hi = jnp.maximum(cur, t)
            t = jnp.minimum(cur, t)
            r[...] = hi

    @pl.when(i == nsteps - 1)
    def _():
        s = jnp.concatenate([r[...] for r in m_refs], axis=1)
        sb = jax.lax.bitcast_convert_type(s, jnp.int32)
        cur = jnp.zeros((s.shape[0], 1), jnp.int32)
        for bit in range(30, -1, -1):
            cand = cur | (1 << bit)
            cnt = jnp.sum((sb >= cand).astype(jnp.int32), axis=1,
                          keepdims=True)
            cur = jnp.where(cnt >= TOPK, cand, cur)
        tau_ref[...] = jnp.broadcast_to(cur, tau_ref.shape)


def _tau(xb, We, be):
    M, K = xb.shape
    H = We.shape[1]
    grid = (H // _NT,)
    return pl.pallas_call(
        _tau_kernel,
        grid=grid,
        in_specs=[
            pl.BlockSpec((M, K), lambda i: (0, 0)),
            pl.BlockSpec((K, _NT), lambda i: (0, i)),
            pl.BlockSpec((1, _NT), lambda i: (0, i)),
        ],
        out_specs=pl.BlockSpec((M, 128), lambda i: (0, 0)),
        out_shape=jax.ShapeDtypeStruct((M, 128), jnp.int32),
        scratch_shapes=[pltpu.VMEM((M, _NBINS), jnp.float32)
                        for _ in range(_P)],
        compiler_params=pltpu.CompilerParams(
            dimension_semantics=("arbitrary",),
        ),
    )(xb, We, be.reshape(1, H))


def _zdec_kernel(x_ref, we_ref, be_ref, tau_ref, wd_ref, bd_ref,
                 z_ref, rec_ref):
    k = pl.program_id(0)
    nsteps = pl.num_programs(0)
    act = _enc_tile(x_ref, we_ref, be_ref)
    tau = tau_ref[:, 0:1]
    ab = jax.lax.bitcast_convert_type(act, jnp.int32)
    z = jnp.where(ab >= tau, act, 0.0)
    z_ref[...] = z
    zb = z.astype(jnp.bfloat16)
    n = rec_ref.shape[1]
    nt = 512
    for j in range(n // nt):
        wb = wd_ref[:, j * nt:(j + 1) * nt].astype(jnp.bfloat16)
        acc = jnp.dot(zb, wb, preferred_element_type=jnp.float32)

        @pl.when(k == 0)
        def _():
            rec_ref[:, j * nt:(j + 1) * nt] = (
                acc + bd_ref[:, j * nt:(j + 1) * nt])

        @pl.when(k > 0)
        def _():
            rec_ref[:, j * nt:(j + 1) * nt] += acc


def _z_and_rec(xb, We, be, tau, Wd, bd):
    M, K = xb.shape
    H = We.shape[1]
    N = Wd.shape[1]
    grid = (H // _NT,)
    return pl.pallas_call(
        _zdec_kernel,
        grid=grid,
        in_specs=[
            pl.BlockSpec((M, K), lambda k: (0, 0)),
            pl.BlockSpec((K, _NT), lambda k: (0, k)),
            pl.BlockSpec((1, _NT), lambda k: (0, k)),
            pl.BlockSpec((M, 128), lambda k: (0, 0)),
            pl.BlockSpec((_NT, N), lambda k: (k, 0)),
            pl.BlockSpec((1, N), lambda k: (0, 0)),
        ],
        out_specs=[
            pl.BlockSpec((M, _NT), lambda k: (0, k)),
            pl.BlockSpec((M, N), lambda k: (0, 0)),
        ],
        out_shape=[
            jax.ShapeDtypeStruct((M, H), jnp.float32),
            jax.ShapeDtypeStruct((M, N), jnp.float32),
        ],
        compiler_params=pltpu.CompilerParams(
            dimension_semantics=("arbitrary",),
        ),
    )(xb, We, be.reshape(1, H), tau, Wd, bd.reshape(1, N))


def kernel(x, We, be, Wd, bd):
    xb = x.astype(jnp.bfloat16)
    tau = _tau(xb, We, be)
    z, rec = _z_and_rec(xb, We, be, tau, Wd, bd)
    return (rec, z)


# speedup vs baseline: 1.3399x; 1.3399x over previous
import jax
import jax.numpy as jnp
from jax.experimental import pallas as pl
from jax.experimental.pallas import tpu as pltpu

TOPK = 64


def _enc_kernel(x_ref, we_ref, be_ref, act_ref):
    xb = x_ref[...]
    wb = we_ref[...].astype(jnp.bfloat16)
    acc = jnp.dot(xb, wb, preferred_element_type=jnp.float32)
    act_ref[...] = jnp.maximum(acc + be_ref[...], 0.0)


def _encoder(xb, We, be):
    M, K = xb.shape
    H = We.shape[1]
    NT = 1024
    grid = (H // NT,)
    return pl.pallas_call(
        _enc_kernel,
        grid=grid,
        in_specs=[
            pl.BlockSpec((M, K), lambda i: (0, 0)),
            pl.BlockSpec((K, NT), lambda i: (0, i)),
            pl.BlockSpec((1, NT), lambda i: (0, i)),
        ],
        out_specs=pl.BlockSpec((M, NT), lambda i: (0, i)),
        out_shape=jax.ShapeDtypeStruct((M, H), jnp.float32),
        compiler_params=pltpu.CompilerParams(
            dimension_semantics=("parallel",),
        ),
    )(xb, We, be.reshape(1, H))


def _topk_kernel(act_ref, wd_ref, z_ref, zb_ref, wdb_ref):
    a = act_ref[...]
    bt, h = a.shape
    nslice = h // 512
    m = [jnp.zeros((bt, 512), jnp.float32) for _ in range(4)]
    for k in range(nslice):
        t = a[:, k * 512:(k + 1) * 512]
        for i in range(4):
            hi = jnp.maximum(m[i], t)
            t = jnp.minimum(m[i], t)
            m[i] = hi
    s = jnp.concatenate(m, axis=1)
    sb = jax.lax.bitcast_convert_type(s, jnp.int32)
    cur = jnp.zeros((bt, 1), jnp.int32)
    for bit in range(30, -1, -1):
        cand = cur | (1 << bit)
        cnt = jnp.sum((sb >= cand).astype(jnp.int32), axis=1, keepdims=True)
        cur = jnp.where(cnt >= TOPK, cand, cur)
    ab = jax.lax.bitcast_convert_type(a, jnp.int32)
    z = jnp.where(ab >= cur, a, 0.0)
    z_ref[...] = z
    zb_ref[...] = z.astype(jnp.bfloat16)
    wdb_ref[...] = wd_ref[...].astype(jnp.bfloat16)


def _topk_z(act, Wd):
    B, H = act.shape
    N = Wd.shape[1]
    BT = 64
    grid = (B // BT,)
    WT = H // (B // BT)
    return pl.pallas_call(
        _topk_kernel,
        grid=grid,
        in_specs=[
            pl.BlockSpec((BT, H), lambda i: (i, 0)),
            pl.BlockSpec((WT, N), lambda i: (i, 0)),
        ],
        out_specs=[
            pl.BlockSpec((BT, H), lambda i: (i, 0)),
            pl.BlockSpec((BT, H), lambda i: (i, 0)),
            pl.BlockSpec((WT, N), lambda i: (i, 0)),
        ],
        out_shape=[
            jax.ShapeDtypeStruct((B, H), jnp.float32),
            jax.ShapeDtypeStruct((B, H), jnp.bfloat16),
            jax.ShapeDtypeStruct((H, N), jnp.bfloat16),
        ],
        compiler_params=pltpu.CompilerParams(
            dimension_semantics=("parallel",),
        ),
    )(act, Wd)


def _dec_kernel(zb_ref, wd_ref, bd_ref, out_ref):
    k = pl.program_id(0)
    zb = zb_ref[...]
    wb = wd_ref[...]
    acc = jnp.dot(zb, wb, preferred_element_type=jnp.float32)

    @pl.when(k == 0)
    def _():
        out_ref[...] = acc + bd_ref[...]

    @pl.when(k > 0)
    def _():
        out_ref[...] += acc


def _decoder(zb, Wdb, bd):
    M, H = zb.shape
    N = Wdb.shape[1]
    KT = 1024
    grid = (H // KT,)
    return pl.pallas_call(
        _dec_kernel,
        grid=grid,
        in_specs=[
            pl.BlockSpec((M, KT), lambda k: (0, k)),
            pl.BlockSpec((KT, N), lambda k: (k, 0)),
            pl.BlockSpec((1, N), lambda k: (0, 0)),
        ],
        out_specs=pl.BlockSpec((M, N), lambda k: (0, 0)),
        out_shape=jax.ShapeDtypeStruct((M, N), jnp.float32),
        compiler_params=pltpu.CompilerParams(
            dimension_semantics=("arbitrary",),
        ),
    )(zb, Wdb, bd.reshape(1, N))


def kernel(x, We, be, Wd, bd):
    xb = x.astype(jnp.bfloat16)
    act = _encoder(xb, We, be)
    z, zb, wdb = _topk_z(act, Wd)
    rec = _decoder(zb, wdb, bd)
    return (rec, z)


# speedup vs baseline: 1.4776x; 1.1028x over previous
import functools

import jax
import jax.numpy as jnp
from jax.experimental import pallas as pl
from jax.experimental.pallas import tpu as pltpu

TOPK = 64


def _enc_kernel(x_ref, we_ref, be_ref, act_ref):
    xb = x_ref[...]
    wb = we_ref[...].astype(jnp.bfloat16)
    acc = jnp.dot(xb, wb, preferred_element_type=jnp.float32)
    act_ref[...] = jnp.maximum(acc + be_ref[...], 0.0)


def _encoder(xb, We, be):
    M, K = xb.shape
    H = We.shape[1]
    NT = 1024
    grid = (H // NT,)
    return pl.pallas_call(
        _enc_kernel,
        grid=grid,
        in_specs=[
            pl.BlockSpec((M, K), lambda i: (0, 0)),
            pl.BlockSpec((K, NT), lambda i: (0, i)),
            pl.BlockSpec((1, NT), lambda i: (0, i)),
        ],
        out_specs=pl.BlockSpec((M, NT), lambda i: (0, i)),
        out_shape=jax.ShapeDtypeStruct((M, H), jnp.float32),
        compiler_params=pltpu.CompilerParams(
            dimension_semantics=("parallel",),
        ),
    )(xb, We, be.reshape(1, H))


def _topk_kernel(act_ref, z_ref, zb_ref):
    a = act_ref[...]
    bt, h = a.shape
    nslice = h // 512
    m = [jnp.zeros((bt, 512), jnp.float32) for _ in range(4)]
    for k in range(nslice):
        t = a[:, k * 512:(k + 1) * 512]
        for i in range(4):
            hi = jnp.maximum(m[i], t)
            t = jnp.minimum(m[i], t)
            m[i] = hi
    s = jnp.concatenate(m, axis=1)
    sb = jax.lax.bitcast_convert_type(s, jnp.int32)
    cur = jnp.zeros((bt, 1), jnp.int32)
    for bit in range(30, -1, -1):
        cand = cur | (1 << bit)
        cnt = jnp.sum((sb >= cand).astype(jnp.int32), axis=1, keepdims=True)
        cur = jnp.where(cnt >= TOPK, cand, cur)
    ab = jax.lax.bitcast_convert_type(a, jnp.int32)
    z = jnp.where(ab >= cur, a, 0.0)
    z_ref[...] = z
    zb_ref[...] = z.astype(jnp.bfloat16)


def _topk_z(act):
    B, H = act.shape
    BT = 128
    grid = (B // BT,)
    return pl.pallas_call(
        _topk_kernel,
        grid=grid,
        in_specs=[pl.BlockSpec((BT, H), lambda i: (i, 0))],
        out_specs=[
            pl.BlockSpec((BT, H), lambda i: (i, 0)),
            pl.BlockSpec((BT, H), lambda i: (i, 0)),
        ],
        out_shape=[
            jax.ShapeDtypeStruct((B, H), jnp.float32),
            jax.ShapeDtypeStruct((B, H), jnp.bfloat16),
        ],
        compiler_params=pltpu.CompilerParams(
            dimension_semantics=("parallel",),
        ),
    )(act)


def _dec_kernel(zb_ref, wd_ref, bd_ref, out_ref):
    k = pl.program_id(0)
    zb = zb_ref[...]
    wb = wd_ref[...].astype(jnp.bfloat16)
    acc = jnp.dot(zb, wb, preferred_element_type=jnp.float32)

    @pl.when(k == 0)
    def _():
        out_ref[...] = acc + bd_ref[...]

    @pl.when(k > 0)
    def _():
        out_ref[...] += acc


def _decoder(zb, Wd, bd):
    M, H = zb.shape
    N = Wd.shape[1]
    KT = 1024
    grid = (H // KT,)
    return pl.pallas_call(
        _dec_kernel,
        grid=grid,
        in_specs=[
            pl.BlockSpec((M, KT), lambda k: (0, k)),
            pl.BlockSpec((KT, N), lambda k: (k, 0)),
            pl.BlockSpec((1, N), lambda k: (0, 0)),
        ],
        out_specs=pl.BlockSpec((M, N), lambda k: (0, 0)),
        out_shape=jax.ShapeDtypeStruct((M, N), jnp.float32),
        compiler_params=pltpu.CompilerParams(
            dimension_semantics=("arbitrary",),
        ),
    )(zb, Wd, bd.reshape(1, N))


def kernel(x, We, be, Wd, bd):
    xb = x.astype(jnp.bfloat16)
    act = _encoder(xb, We, be)
    z, zb = _topk_z(act)
    rec = _decoder(zb, Wd, bd)
    return (rec, z)
